# baseline (speedup 1.0000x reference)
import functools

import jax
import jax.numpy as jnp
import numpy as np
from jax.experimental import pallas as pl
from jax.experimental.pallas import tpu as pltpu

CIN = 96
COUT = 96
NCB = 16
K = 16
SUB = 54
CPB = CIN // NCB
OH = 112
OW = 112
NL = NCB * K
TH = 8
NT = OH // TH
NW = TH * OW


def _body(params_ref, xc_ref, dd_ref, cblkt_ref, t2k_ref, den_ref,
          lutt_ref, bias_ref, out_ref):
    t = pl.program_id(1)
    x_sc = params_ref[3]
    x_zp = params_ref[4]

    slab = xc_ref[0, :, pl.ds(16 * t, 16), :]
    sbf = ((slab.astype(jnp.float32) - x_zp) * x_sc).astype(jnp.bfloat16)
    eo_all = jax.lax.dot_general(
        sbf.reshape(CIN * 16, 224), dd_ref[...], (((1,), (0,)), ((), ())),
        preferred_element_type=jnp.float32).astype(jnp.bfloat16)
    eo3 = eo_all.reshape(CIN, 16, 224)

    r32 = xc_ref[0, :, jnp.maximum(16 * t - 1, 0), :]
    w = jnp.where(t > 0, x_sc, 0.0)
    rbf = ((r32.astype(jnp.float32) - x_zp) * w).astype(jnp.bfloat16)
    eo_m1 = jax.lax.dot_general(
        rbf, dd_ref[...], (((1,), (0,)), ((), ())),
        preferred_element_type=jnp.float32).astype(jnp.bfloat16)

    pee = jnp.concatenate([eo3[:, 2 * rr, :OW] for rr in range(TH)], axis=1)
    peo = jnp.concatenate([eo3[:, 2 * rr, OW:] for rr in range(TH)], axis=1)
    poe = jnp.concatenate(
        [eo3[:, 2 * rr + 1, :OW] for rr in range(TH)], axis=1)
    poo = jnp.concatenate(
        [eo3[:, 2 * rr + 1, OW:] for rr in range(TH)], axis=1)
    qe = jnp.concatenate([eo_m1[:, :OW], poe[:, :NW - OW]], axis=1)
    qo = jnp.concatenate([eo_m1[:, OW:], poo[:, :NW - OW]], axis=1)

    lane = jax.lax.broadcasted_iota(jnp.int32, (CIN, NW), 1)
    seg0 = (lane % OW) == 0

    def mshift(p):
        sh = jnp.concatenate([jnp.zeros((CIN, 1), jnp.bfloat16),
                              p[:, :NW - 1]], axis=1)
        return jnp.where(seg0, jnp.bfloat16(0), sh)

    s_acc = jnp.zeros((NL, NW), jnp.float32)
    taps = (mshift(qo), qe, qo,
            mshift(peo), pee, peo,
            mshift(poo), poe, poo)
    for pos, tap in enumerate(taps):
        cp = cblkt_ref[:, pos * CIN:(pos + 1) * CIN]
        s_acc = s_acc + jax.lax.dot_general(
            cp, tap, (((1,), (0,)), ((), ())),
            preferred_element_type=jnp.float32)

    r = jnp.round(s_acc / den_ref[:, 0:1])
    key = t2k_ref[:, 0:1] - 32 * r.astype(jnp.int32)

    sub = jax.lax.broadcasted_iota(jnp.int32, (NL, NW), 0)
    m = key
    for sh in (1, 2, 4, 8):
        up = jnp.roll(m, -sh, axis=0)
        dn = jnp.roll(m, sh, axis=0)
        m = jnp.minimum(m, jnp.where((sub & sh) == 0, up, dn))

    onehot = (key == m).astype(jnp.bfloat16)
    acc = jax.lax.dot_general(
        lutt_ref[...], onehot, (((1,), (0,)), ((), ())),
        preferred_element_type=jnp.float32)

    lut_s = params_ref[0]
    out_s = params_ref[1]
    out_z = params_ref[2]
    outf = acc * lut_s + bias_ref[:, 0:1]
    outf = jnp.maximum(outf, 0.0)
    q = jnp.clip(outf / out_s + out_z, -128.0, 127.0)
    q8 = jnp.round(q).astype(jnp.int8)
    out_ref[...] = q8.reshape(1, COUT, 1, 1, NW)


@functools.partial(jax.jit, static_argnames=())
def kernel(x_q, x_s, x_z, centroids_q, centroids_s, centroids_z,
           lut_q, lut_s, lut_z, bias_q, bias_s, bias_z, output_s, output_z):
    b = x_q.shape[0]

    dd = np.zeros((224, 224), np.float32)
    dd[np.arange(0, 224, 2), np.arange(112)] = 1.0
    dd[np.arange(1, 224, 2), np.arange(112, 224)] = 1.0
    dd_bf = jnp.asarray(dd, dtype=jnp.bfloat16)

    cq = (centroids_q - centroids_z).astype(jnp.float32)
    c_dq = cq * centroids_s
    cbi = np.arange(NCB)[:, None, None]
    kii = np.arange(K)[None, :, None]
    sii = np.arange(SUB)[None, None, :]
    dprime = (sii % 9) * CIN + (cbi * CPB + sii // 9)
    col = cbi * K + kii
    dprime = np.broadcast_to(dprime, (NCB, K, SUB))
    col = np.broadcast_to(col, (NCB, K, SUB))
    cblkt = jnp.zeros((NL, 9 * CIN), jnp.float32).at[col, dprime].set(c_dq)
    cblkt_bf = cblkt.astype(jnp.bfloat16)

    y2 = jnp.sum(c_dq * c_dq, axis=-1)
    den = x_s[0] * centroids_s[:, 0, 0]
    t2 = jnp.round(y2 / den[:, None])
    t2k = (16 * t2.astype(jnp.int32)
           + jnp.arange(K, dtype=jnp.int32)[None, :]).reshape(NL, 1)
    den_c = jnp.broadcast_to(den[:, None], (NCB, K)).reshape(NL, 1)

    lutt_bf = (lut_q - lut_z[0]).astype(jnp.bfloat16).reshape(NL, COUT).T
    bias_f = ((bias_q - bias_z[0]).astype(jnp.float32) * bias_s[0]
              ).reshape(COUT, 1)
    params = jnp.stack([lut_s[0], output_s[0],
                        output_z[0].astype(jnp.float32), x_s[0],
                        x_z[0].astype(jnp.float32)])

    out5 = pl.pallas_call(
        _body,
        grid=(b, NT),
        in_specs=[
            pl.BlockSpec(memory_space=pltpu.SMEM),
            pl.BlockSpec((1, CIN, 224, 224), lambda bb, tt: (bb, 0, 0, 0)),
            pl.BlockSpec((224, 224), lambda bb, tt: (0, 0)),
            pl.BlockSpec((NL, 9 * CIN), lambda bb, tt: (0, 0)),
            pl.BlockSpec((NL, 1), lambda bb, tt: (0, 0)),
            pl.BlockSpec((NL, 1), lambda bb, tt: (0, 0)),
            pl.BlockSpec((COUT, NL), lambda bb, tt: (0, 0)),
            pl.BlockSpec((COUT, 1), lambda bb, tt: (0, 0)),
        ],
        out_specs=pl.BlockSpec((1, COUT, 1, 1, NW),
                               lambda bb, tt: (bb, 0, tt, 0, 0)),
        out_shape=jax.ShapeDtypeStruct((b, COUT, NT, 1, NW), jnp.int8),
        compiler_params=pltpu.CompilerParams(
            dimension_semantics=("arbitrary", "arbitrary")),
    )(params, x_q, dd_bf, cblkt_bf, t2k, den_c, lutt_bf, bias_f)
    return out5.reshape(b, COUT, OH, OW)


# speedup vs baseline: 1.0303x; 1.0303x over previous
import functools

import jax
import jax.numpy as jnp
import numpy as np
from jax.experimental import pallas as pl
from jax.experimental.pallas import tpu as pltpu

CIN = 96
COUT = 96
NCB = 16
K = 16
SUB = 54
CPB = CIN // NCB
OH = 112
OW = 112
NL = NCB * K
TH = 8
NT = OH // TH
NW = TH * OW


def _body(params_ref, xc_ref, xp_ref, dd_ref, cblkt_ref, t2k_ref, den_ref,
          lutt_ref, bias_ref, out_ref):
    t = pl.program_id(1)
    x_sc = params_ref[3]
    x_zp = params_ref[4]

    slab = xc_ref[0]
    sbf = ((slab.astype(jnp.float32) - x_zp) * x_sc).astype(jnp.bfloat16)
    eo_all = jax.lax.dot_general(
        sbf.reshape(CIN * 16, 224), dd_ref[...], (((1,), (0,)), ((), ())),
        preferred_element_type=jnp.float32).astype(jnp.bfloat16)
    eo3 = eo_all.reshape(CIN, 16, 224)

    r32 = xp_ref[0, :, 7, :]
    w = jnp.where(t > 0, x_sc, 0.0)
    rbf = ((r32.astype(jnp.float32) - x_zp) * w).astype(jnp.bfloat16)
    eo_m1 = jax.lax.dot_general(
        rbf, dd_ref[...], (((1,), (0,)), ((), ())),
        preferred_element_type=jnp.float32).astype(jnp.bfloat16)

    pee = jnp.concatenate([eo3[:, 2 * rr, :OW] for rr in range(TH)], axis=1)
    peo = jnp.concatenate([eo3[:, 2 * rr, OW:] for rr in range(TH)], axis=1)
    poe = jnp.concatenate(
        [eo3[:, 2 * rr + 1, :OW] for rr in range(TH)], axis=1)
    poo = jnp.concatenate(
        [eo3[:, 2 * rr + 1, OW:] for rr in range(TH)], axis=1)
    qe = jnp.concatenate([eo_m1[:, :OW], poe[:, :NW - OW]], axis=1)
    qo = jnp.concatenate([eo_m1[:, OW:], poo[:, :NW - OW]], axis=1)

    lane = jax.lax.broadcasted_iota(jnp.int32, (CIN, NW), 1)
    seg0 = (lane % OW) == 0

    def mshift(p):
        sh = jnp.concatenate([jnp.zeros((CIN, 1), jnp.bfloat16),
                              p[:, :NW - 1]], axis=1)
        return jnp.where(seg0, jnp.bfloat16(0), sh)

    s_acc = jnp.zeros((NL, NW), jnp.float32)
    taps = (mshift(qo), qe, qo,
            mshift(peo), pee, peo,
            mshift(poo), poe, poo)
    for pos, tap in enumerate(taps):
        cp = cblkt_ref[:, pos * CIN:(pos + 1) * CIN]
        s_acc = s_acc + jax.lax.dot_general(
            cp, tap, (((1,), (0,)), ((), ())),
            preferred_element_type=jnp.float32)

    r = jnp.round(s_acc / den_ref[:, 0:1])
    key = t2k_ref[:, 0:1] - 32 * r.astype(jnp.int32)

    sub = jax.lax.broadcasted_iota(jnp.int32, (NL, NW), 0)
    m = key
    for sh in (1, 2, 4, 8):
        up = jnp.roll(m, -sh, axis=0)
        dn = jnp.roll(m, sh, axis=0)
        m = jnp.minimum(m, jnp.where((sub & sh) == 0, up, dn))

    onehot = (key == m).astype(jnp.bfloat16)
    acc = jax.lax.dot_general(
        lutt_ref[...], onehot, (((1,), (0,)), ((), ())),
        preferred_element_type=jnp.float32)

    lut_s = params_ref[0]
    out_s = params_ref[1]
    out_z = params_ref[2]
    outf = acc * lut_s + bias_ref[:, 0:1]
    outf = jnp.maximum(outf, 0.0)
    q = jnp.clip(outf / out_s + out_z, -128.0, 127.0)
    q8 = jnp.round(q).astype(jnp.int8)
    out_ref[...] = q8.reshape(1, COUT, 1, 1, NW)


@functools.partial(jax.jit, static_argnames=())
def kernel(x_q, x_s, x_z, centroids_q, centroids_s, centroids_z,
           lut_q, lut_s, lut_z, bias_q, bias_s, bias_z, output_s, output_z):
    b = x_q.shape[0]

    dd = np.zeros((224, 224), np.float32)
    dd[np.arange(0, 224, 2), np.arange(112)] = 1.0
    dd[np.arange(1, 224, 2), np.arange(112, 224)] = 1.0
    dd_bf = jnp.asarray(dd, dtype=jnp.bfloat16)

    cq = (centroids_q - centroids_z).astype(jnp.float32)
    c_dq = cq * centroids_s
    cbi = np.arange(NCB)[:, None, None]
    kii = np.arange(K)[None, :, None]
    sii = np.arange(SUB)[None, None, :]
    dprime = (sii % 9) * CIN + (cbi * CPB + sii // 9)
    col = cbi * K + kii
    dprime = np.broadcast_to(dprime, (NCB, K, SUB))
    col = np.broadcast_to(col, (NCB, K, SUB))
    cblkt = jnp.zeros((NL, 9 * CIN), jnp.float32).at[col, dprime].set(c_dq)
    cblkt_bf = cblkt.astype(jnp.bfloat16)

    y2 = jnp.sum(c_dq * c_dq, axis=-1)
    den = x_s[0] * centroids_s[:, 0, 0]
    t2 = jnp.round(y2 / den[:, None])
    t2k = (16 * t2.astype(jnp.int32)
           + jnp.arange(K, dtype=jnp.int32)[None, :]).reshape(NL, 1)
    den_c = jnp.broadcast_to(den[:, None], (NCB, K)).reshape(NL, 1)

    lutt_bf = (lut_q - lut_z[0]).astype(jnp.bfloat16).reshape(NL, COUT).T
    bias_f = ((bias_q - bias_z[0]).astype(jnp.float32) * bias_s[0]
              ).reshape(COUT, 1)
    params = jnp.stack([lut_s[0], output_s[0],
                        output_z[0].astype(jnp.float32), x_s[0],
                        x_z[0].astype(jnp.float32)])

    out5 = pl.pallas_call(
        _body,
        grid=(b, NT),
        in_specs=[
            pl.BlockSpec(memory_space=pltpu.SMEM),
            pl.BlockSpec((1, CIN, 16, 224), lambda bb, tt: (bb, 0, tt, 0)),
            pl.BlockSpec((1, CIN, 8, 224),
                         lambda bb, tt: (bb, 0, jnp.maximum(2 * tt - 1, 0), 0)),
            pl.BlockSpec((224, 224), lambda bb, tt: (0, 0)),
            pl.BlockSpec((NL, 9 * CIN), lambda bb, tt: (0, 0)),
            pl.BlockSpec((NL, 1), lambda bb, tt: (0, 0)),
            pl.BlockSpec((NL, 1), lambda bb, tt: (0, 0)),
            pl.BlockSpec((COUT, NL), lambda bb, tt: (0, 0)),
            pl.BlockSpec((COUT, 1), lambda bb, tt: (0, 0)),
        ],
        out_specs=pl.BlockSpec((1, COUT, 1, 1, NW),
                               lambda bb, tt: (bb, 0, tt, 0, 0)),
        out_shape=jax.ShapeDtypeStruct((b, COUT, NT, 1, NW), jnp.int8),
        compiler_params=pltpu.CompilerParams(
            dimension_semantics=("arbitrary", "arbitrary")),
    )(params, x_q, x_q, dd_bf, cblkt_bf, t2k, den_c, lutt_bf, bias_f)
    return out5.reshape(b, COUT, OH, OW)


# speedup vs baseline: 1.0342x; 1.0038x over previous
import functools

import jax
import jax.numpy as jnp
import numpy as np
from jax.experimental import pallas as pl
from jax.experimental.pallas import tpu as pltpu

CIN = 96
COUT = 96
NCB = 16
K = 16
SUB = 54
CPB = CIN // NCB
OH = 112
OW = 112
NL = NCB * K
TH = 8
NT = OH // TH
NW = TH * OW


def _body(params_ref, xc_ref, dd_ref, cblkt_ref, t2k_ref, den_ref,
          lutt_ref, bias_ref, out_ref, prev_ref):
    t = pl.program_id(1)
    x_sc = params_ref[3]
    x_zp = params_ref[4]

    slab = xc_ref[0]
    sbf = ((slab.astype(jnp.float32) - x_zp) * x_sc).astype(jnp.bfloat16)
    eo_all = jax.lax.dot_general(
        sbf.reshape(CIN * 16, 224), dd_ref[...], (((1,), (0,)), ((), ())),
        preferred_element_type=jnp.float32).astype(jnp.bfloat16)
    eo3 = eo_all.reshape(CIN, 16, 224)

    eo_m1 = jnp.where(t > 0, prev_ref[...], jnp.bfloat16(0))
    prev_ref[...] = eo3[:, 15, :]

    pee = jnp.concatenate([eo3[:, 2 * rr, :OW] for rr in range(TH)], axis=1)
    peo = jnp.concatenate([eo3[:, 2 * rr, OW:] for rr in range(TH)], axis=1)
    poe = jnp.concatenate(
        [eo3[:, 2 * rr + 1, :OW] for rr in range(TH)], axis=1)
    poo = jnp.concatenate(
        [eo3[:, 2 * rr + 1, OW:] for rr in range(TH)], axis=1)
    qe = jnp.concatenate([eo_m1[:, :OW], poe[:, :NW - OW]], axis=1)
    qo = jnp.concatenate([eo_m1[:, OW:], poo[:, :NW - OW]], axis=1)

    lane = jax.lax.broadcasted_iota(jnp.int32, (CIN, NW), 1)
    seg0 = (lane % OW) == 0

    def mshift(p):
        sh = jnp.concatenate([jnp.zeros((CIN, 1), jnp.bfloat16),
                              p[:, :NW - 1]], axis=1)
        return jnp.where(seg0, jnp.bfloat16(0), sh)

    s_acc = jnp.zeros((NL, NW), jnp.float32)
    taps = (mshift(qo), qe, qo,
            mshift(peo), pee, peo,
            mshift(poo), poe, poo)
    for pos, tap in enumerate(taps):
        cp = cblkt_ref[:, pos * CIN:(pos + 1) * CIN]
        s_acc = s_acc + jax.lax.dot_general(
            cp, tap, (((1,), (0,)), ((), ())),
            preferred_element_type=jnp.float32)

    r = jnp.round(s_acc / den_ref[:, 0:1])
    key = t2k_ref[:, 0:1] - 32 * r.astype(jnp.int32)

    sub = jax.lax.broadcasted_iota(jnp.int32, (NL, NW), 0)
    m = key
    for sh in (1, 2, 4, 8):
        up = jnp.roll(m, -sh, axis=0)
        dn = jnp.roll(m, sh, axis=0)
        m = jnp.minimum(m, jnp.where((sub & sh) == 0, up, dn))

    onehot = (key == m).astype(jnp.bfloat16)
    acc = jax.lax.dot_general(
        lutt_ref[...], onehot, (((1,), (0,)), ((), ())),
        preferred_element_type=jnp.float32)

    lut_s = params_ref[0]
    out_s = params_ref[1]
    out_z = params_ref[2]
    outf = acc * lut_s + bias_ref[:, 0:1]
    outf = jnp.maximum(outf, 0.0)
    q = jnp.clip(outf / out_s + out_z, -128.0, 127.0)
    q8 = jnp.round(q).astype(jnp.int8)
    out_ref[...] = q8.reshape(1, COUT, 1, 1, NW)


@functools.partial(jax.jit, static_argnames=())
def kernel(x_q, x_s, x_z, centroids_q, centroids_s, centroids_z,
           lut_q, lut_s, lut_z, bias_q, bias_s, bias_z, output_s, output_z):
    b = x_q.shape[0]

    dd = np.zeros((224, 224), np.float32)
    dd[np.arange(0, 224, 2), np.arange(112)] = 1.0
    dd[np.arange(1, 224, 2), np.arange(112, 224)] = 1.0
    dd_bf = jnp.asarray(dd, dtype=jnp.bfloat16)

    cq = (centroids_q - centroids_z).astype(jnp.float32)
    c_dq = cq * centroids_s
    cbi = np.arange(NCB)[:, None, None]
    kii = np.arange(K)[None, :, None]
    sii = np.arange(SUB)[None, None, :]
    dprime = (sii % 9) * CIN + (cbi * CPB + sii // 9)
    col = cbi * K + kii
    dprime = np.broadcast_to(dprime, (NCB, K, SUB))
    col = np.broadcast_to(col, (NCB, K, SUB))
    cblkt = jnp.zeros((NL, 9 * CIN), jnp.float32).at[col, dprime].set(c_dq)
    cblkt_bf = cblkt.astype(jnp.bfloat16)

    y2 = jnp.sum(c_dq * c_dq, axis=-1)
    den = x_s[0] * centroids_s[:, 0, 0]
    t2 = jnp.round(y2 / den[:, None])
    t2k = (16 * t2.astype(jnp.int32)
           + jnp.arange(K, dtype=jnp.int32)[None, :]).reshape(NL, 1)
    den_c = jnp.broadcast_to(den[:, None], (NCB, K)).reshape(NL, 1)

    lutt_bf = (lut_q - lut_z[0]).astype(jnp.bfloat16).reshape(NL, COUT).T
    bias_f = ((bias_q - bias_z[0]).astype(jnp.float32) * bias_s[0]
              ).reshape(COUT, 1)
    params = jnp.stack([lut_s[0], output_s[0],
                        output_z[0].astype(jnp.float32), x_s[0],
                        x_z[0].astype(jnp.float32)])

    out5 = pl.pallas_call(
        _body,
        grid=(b, NT),
        in_specs=[
            pl.BlockSpec(memory_space=pltpu.SMEM),
            pl.BlockSpec((1, CIN, 16, 224), lambda bb, tt: (bb, 0, tt, 0)),
            pl.BlockSpec((224, 224), lambda bb, tt: (0, 0)),
            pl.BlockSpec((NL, 9 * CIN), lambda bb, tt: (0, 0)),
            pl.BlockSpec((NL, 1), lambda bb, tt: (0, 0)),
            pl.BlockSpec((NL, 1), lambda bb, tt: (0, 0)),
            pl.BlockSpec((COUT, NL), lambda bb, tt: (0, 0)),
            pl.BlockSpec((COUT, 1), lambda bb, tt: (0, 0)),
        ],
        out_specs=pl.BlockSpec((1, COUT, 1, 1, NW),
                               lambda bb, tt: (bb, 0, tt, 0, 0)),
        out_shape=jax.ShapeDtypeStruct((b, COUT, NT, 1, NW), jnp.int8),
        scratch_shapes=[pltpu.VMEM((CIN, 224), jnp.bfloat16)],
        compiler_params=pltpu.CompilerParams(
            dimension_semantics=("arbitrary", "arbitrary")),
    )(params, x_q, dd_bf, cblkt_bf, t2k, den_c, lutt_bf, bias_f)
    return out5.reshape(b, COUT, OH, OW)


# speedup vs baseline: 1.2475x; 1.2063x over previous
import functools

import jax
import jax.numpy as jnp
import numpy as np
from jax.experimental import pallas as pl
from jax.experimental.pallas import tpu as pltpu

CIN = 96
COUT = 96
NCB = 16
K = 16
SUB = 54
CPB = CIN // NCB
OH = 112
OW = 112
NL = NCB * K
TH = 8
NT = OH // TH
NW = TH * OW


def _body(params_ref, xc_ref, dd_ref, cblkt_ref, t2k_ref, den_ref,
          lutt_ref, bias_ref, out_ref, prev_ref):
    t = pl.program_id(1)
    x_sc = params_ref[3]
    x_zp = params_ref[4]

    slab = xc_ref[0]
    sbf = ((slab.astype(jnp.float32) - x_zp) * x_sc).astype(jnp.bfloat16)
    eo_all = jax.lax.dot_general(
        sbf.reshape(CIN * 16, 224), dd_ref[...], (((1,), (0,)), ((), ())),
        preferred_element_type=jnp.float32).astype(jnp.bfloat16)
    eo3 = eo_all.reshape(CIN, 16, 224)

    eo_m1 = jnp.where(t > 0, prev_ref[...], jnp.bfloat16(0))
    prev_ref[...] = eo3[:, 15, :]

    pee = jnp.concatenate([eo3[:, 2 * rr, :OW] for rr in range(TH)], axis=1)
    peo = jnp.concatenate([eo3[:, 2 * rr, OW:] for rr in range(TH)], axis=1)
    poe = jnp.concatenate(
        [eo3[:, 2 * rr + 1, :OW] for rr in range(TH)], axis=1)
    poo = jnp.concatenate(
        [eo3[:, 2 * rr + 1, OW:] for rr in range(TH)], axis=1)
    qe = jnp.concatenate([eo_m1[:, :OW], poe[:, :NW - OW]], axis=1)
    qo = jnp.concatenate([eo_m1[:, OW:], poo[:, :NW - OW]], axis=1)

    lane = jax.lax.broadcasted_iota(jnp.int32, (CIN, NW), 1)
    seg0 = (lane % OW) == 0

    def mshift(p):
        sh = jnp.concatenate([jnp.zeros((CIN, 1), jnp.bfloat16),
                              p[:, :NW - 1]], axis=1)
        return jnp.where(seg0, jnp.bfloat16(0), sh)

    s_acc = jnp.zeros((NL, NW), jnp.float32)
    taps = (mshift(qo), qe, qo,
            mshift(peo), pee, peo,
            mshift(poo), poe, poo)
    for pos, tap in enumerate(taps):
        cp = cblkt_ref[:, pos * CIN:(pos + 1) * CIN]
        s_acc = s_acc + jax.lax.dot_general(
            cp, tap, (((1,), (0,)), ((), ())),
            preferred_element_type=jnp.float32)

    r = jnp.round(s_acc / den_ref[:, 0:1])
    key = t2k_ref[:, 0:1] - 32 * r.astype(jnp.int32)

    sub = jax.lax.broadcasted_iota(jnp.int32, (NL, NW), 0)
    m = key
    for sh in (1, 2, 4, 8):
        up = jnp.roll(m, -sh, axis=0)
        dn = jnp.roll(m, sh, axis=0)
        m = jnp.minimum(m, jnp.where((sub & sh) == 0, up, dn))

    onehot = (key == m).astype(jnp.bfloat16)
    acc = jax.lax.dot_general(
        lutt_ref[...], onehot, (((1,), (0,)), ((), ())),
        preferred_element_type=jnp.float32)

    lut_s = params_ref[0]
    out_s = params_ref[1]
    out_z = params_ref[2]
    outf = acc * lut_s + bias_ref[:, 0:1]
    outf = jnp.maximum(outf, 0.0)
    q = jnp.clip(outf / out_s + out_z, -128.0, 127.0)
    q8 = jnp.round(q).astype(jnp.int8)
    out_ref[...] = q8.reshape(1, COUT, 1, 1, NW)


@functools.partial(jax.jit, static_argnames=())
def kernel(x_q, x_s, x_z, centroids_q, centroids_s, centroids_z,
           lut_q, lut_s, lut_z, bias_q, bias_s, bias_z, output_s, output_z):
    b = x_q.shape[0]

    dd = np.zeros((224, 224), np.float32)
    dd[np.arange(0, 224, 2), np.arange(112)] = 1.0
    dd[np.arange(1, 224, 2), np.arange(112, 224)] = 1.0
    dd_bf = jnp.asarray(dd, dtype=jnp.bfloat16)

    cq = (centroids_q - centroids_z).astype(jnp.float32)
    c_dq = cq * centroids_s
    place = np.zeros((NCB, SUB, 9 * CIN), np.float32)
    for cb in range(NCB):
        si = np.arange(SUB)
        place[cb, si, (si % 9) * CIN + cb * CPB + si // 9] = 1.0
    cblkt = jnp.einsum('cks,csd->ckd', c_dq, jnp.asarray(place),
                       precision=jax.lax.Precision.HIGHEST)
    cblkt_bf = cblkt.reshape(NL, 9 * CIN).astype(jnp.bfloat16)

    y2 = jnp.sum(c_dq * c_dq, axis=-1)
    den = x_s[0] * centroids_s[:, 0, 0]
    t2 = jnp.round(y2 / den[:, None])
    t2k = (16 * t2.astype(jnp.int32)
           + jnp.arange(K, dtype=jnp.int32)[None, :]).reshape(NL, 1)
    den_c = jnp.broadcast_to(den[:, None], (NCB, K)).reshape(NL, 1)

    lutt_bf = (lut_q - lut_z[0]).astype(jnp.bfloat16).reshape(NL, COUT).T
    bias_f = ((bias_q - bias_z[0]).astype(jnp.float32) * bias_s[0]
              ).reshape(COUT, 1)
    params = jnp.stack([lut_s[0], output_s[0],
                        output_z[0].astype(jnp.float32), x_s[0],
                        x_z[0].astype(jnp.float32)])

    out5 = pl.pallas_call(
        _body,
        grid=(b, NT),
        in_specs=[
            pl.BlockSpec(memory_space=pltpu.SMEM),
            pl.BlockSpec((1, CIN, 16, 224), lambda bb, tt: (bb, 0, tt, 0)),
            pl.BlockSpec((224, 224), lambda bb, tt: (0, 0)),
            pl.BlockSpec((NL, 9 * CIN), lambda bb, tt: (0, 0)),
            pl.BlockSpec((NL, 1), lambda bb, tt: (0, 0)),
            pl.BlockSpec((NL, 1), lambda bb, tt: (0, 0)),
            pl.BlockSpec((COUT, NL), lambda bb, tt: (0, 0)),
            pl.BlockSpec((COUT, 1), lambda bb, tt: (0, 0)),
        ],
        out_specs=pl.BlockSpec((1, COUT, 1, 1, NW),
                               lambda bb, tt: (bb, 0, tt, 0, 0)),
        out_shape=jax.ShapeDtypeStruct((b, COUT, NT, 1, NW), jnp.int8),
        scratch_shapes=[pltpu.VMEM((CIN, 224), jnp.bfloat16)],
        compiler_params=pltpu.CompilerParams(
            dimension_semantics=("arbitrary", "arbitrary")),
    )(params, x_q, dd_bf, cblkt_bf, t2k, den_c, lutt_bf, bias_f)
    return out5.reshape(b, COUT, OH, OW)
